# baseline (speedup 1.0000x reference)
import functools
import math

import jax
import jax.numpy as jnp
from jax import lax
from jax.experimental import pallas as pl
from jax.experimental.pallas import tpu as pltpu
from jax.experimental.pallas import tpu_sc as plsc

VOCAB_SIZE = 1_000_000
EMBED_DIM = 64
BATCH = 16384
HIST = 50
B_TOTAL = BATCH * HIST
NUM_WORKERS = 32
B_PER_W = B_TOTAL // NUM_WORKERS
CHUNK = 128
CHUNKS_PER_W = B_PER_W // CHUNK
SCALE = math.sqrt(EMBED_DIM)
LANES = 16

_mesh = plsc.VectorSubcoreMesh(core_axis_name="c", subcore_axis_name="s")


@functools.partial(
    pl.kernel,
    mesh=_mesh,
    out_type=jax.ShapeDtypeStruct((B_TOTAL, EMBED_DIM), jnp.float32),
    scratch_types=[
        pltpu.VMEM((CHUNKS_PER_W, CHUNK), jnp.int32),
        pltpu.VMEM((CHUNK, EMBED_DIM), jnp.float32),
        pltpu.SemaphoreType.DMA,
    ],
    compiler_params=pltpu.CompilerParams(use_tc_tiling_on_sc=False),
)
def _embed_sc(idx_hbm, table_hbm, out_hbm, idx_v, rows_v, sem):
    c = lax.axis_index("c")
    s = lax.axis_index("s")
    wid = s * 2 + c
    base = wid * B_PER_W

    pltpu.sync_copy(idx_hbm.at[pl.ds(wid * CHUNKS_PER_W, CHUNKS_PER_W)], idx_v)

    def chunk_body(j, carry):
        pltpu.async_copy(table_hbm.at[idx_v.at[j]], rows_v, sem).wait()

        def row_body(i, carry2):
            for d in range(EMBED_DIM // LANES):
                sl = pl.ds(d * LANES, LANES)
                rows_v[i, sl] = rows_v[i, sl] * SCALE
            return carry2

        lax.fori_loop(0, CHUNK, row_body, 0)

        pltpu.sync_copy(rows_v, out_hbm.at[pl.ds(base + j * CHUNK, CHUNK)])
        return carry

    lax.fori_loop(0, CHUNKS_PER_W, chunk_body, 0)


def kernel(x, embed_weight):
    idx = x.reshape(B_TOTAL // CHUNK, CHUNK).astype(jnp.int32)
    out = _embed_sc(idx, embed_weight)
    return out.reshape(BATCH, HIST, EMBED_DIM)


# speedup vs baseline: 1.2086x; 1.2086x over previous
import functools
import math

import jax
import jax.numpy as jnp
from jax import lax
from jax.experimental import pallas as pl
from jax.experimental.pallas import tpu as pltpu
from jax.experimental.pallas import tpu_sc as plsc

VOCAB_SIZE = 1_000_000
EMBED_DIM = 64
BATCH = 16384
HIST = 50
B_TOTAL = BATCH * HIST
NUM_WORKERS = 32
B_PER_W = B_TOTAL // NUM_WORKERS
CHUNK = 128
CHUNKS_PER_W = B_PER_W // CHUNK
NBUF = 8
NGATHER = 4
NOUT = CHUNKS_PER_W // NBUF
SCALE = math.sqrt(EMBED_DIM)
LANES = 16

_mesh = plsc.VectorSubcoreMesh(core_axis_name="c", subcore_axis_name="s")


@functools.partial(
    pl.kernel,
    mesh=_mesh,
    out_type=jax.ShapeDtypeStruct((B_TOTAL, EMBED_DIM), jnp.float32),
    scratch_types=[
        pltpu.VMEM((CHUNKS_PER_W, CHUNK), jnp.int32),
        pltpu.VMEM((NBUF, CHUNK, EMBED_DIM), jnp.float32),
    ]
    + [pltpu.SemaphoreType.DMA] * (2 * NBUF),
    compiler_params=pltpu.CompilerParams(use_tc_tiling_on_sc=False),
)
def _embed_sc(idx_hbm, table_hbm, out_hbm, idx_v, rows_v, *sems):
    gsem = sems[:NBUF]
    wsem = sems[NBUF:]
    c = lax.axis_index("c")
    s = lax.axis_index("s")
    wid = s * 2 + c
    base = wid * B_PER_W

    pltpu.sync_copy(idx_hbm.at[pl.ds(wid * CHUNKS_PER_W, CHUNKS_PER_W)], idx_v)

    def out_slice(j):
        return out_hbm.at[pl.ds(base + j * CHUNK, CHUNK)]

    for j in range(NGATHER):
        pltpu.async_copy(table_hbm.at[idx_v.at[j]], rows_v.at[j], gsem[j])

    def outer(o, carry):
        jbase = o * NBUF
        for b in range(NBUF):
            j = jbase + b
            bn = (b + NGATHER) % NBUF
            jn = j + NGATHER

            if b + NGATHER >= NBUF:
                @pl.when(jn < CHUNKS_PER_W)
                def _wait_then_gather():
                    pltpu.make_async_copy(
                        rows_v.at[bn], out_slice(jn - NBUF), wsem[bn]
                    ).wait()
                    pltpu.async_copy(
                        table_hbm.at[idx_v.at[jn]], rows_v.at[bn], gsem[bn]
                    )
            else:
                @pl.when(jn < CHUNKS_PER_W)
                def _prefetch():
                    @pl.when(jn >= NBUF)
                    def _wait_write():
                        pltpu.make_async_copy(
                            rows_v.at[bn], out_slice(jn - NBUF), wsem[bn]
                        ).wait()

                    pltpu.async_copy(
                        table_hbm.at[idx_v.at[jn]], rows_v.at[bn], gsem[bn]
                    )

            pltpu.make_async_copy(
                table_hbm.at[idx_v.at[j]], rows_v.at[b], gsem[b]
            ).wait()

            rv = rows_v.at[b]

            def row_body(i, carry2):
                for r in range(4):
                    for d in range(EMBED_DIM // LANES):
                        sl = pl.ds(d * LANES, LANES)
                        rv[i * 4 + r, sl] = rv[i * 4 + r, sl] * SCALE
                return carry2

            lax.fori_loop(0, CHUNK // 4, row_body, 0)

            pltpu.async_copy(rv, out_slice(j), wsem[b])
        return carry

    lax.fori_loop(0, NOUT, outer, 0)

    for b in range(NBUF):
        j = CHUNKS_PER_W - NBUF + b
        pltpu.make_async_copy(rows_v.at[b], out_slice(j), wsem[b]).wait()


def kernel(x, embed_weight):
    idx = x.reshape(B_TOTAL // CHUNK, CHUNK).astype(jnp.int32)
    out = _embed_sc(idx, embed_weight)
    return out.reshape(BATCH, HIST, EMBED_DIM)
